# Initial kernel scaffold; baseline (speedup 1.0000x reference)
import jax
import jax.numpy as jnp
from jax.experimental import pallas as pl


def kernel(x, edge_index_x, y, edge_index_y, Wx1, bx1, Wx2, bx2, fcWx, fcbx, Wy1, by1, Wy2, by2, fcWy, fcby):
    raise NotImplementedError("write your pallas kernel here")


# speedup vs baseline: 9.3092x; 9.3092x over previous
import functools

import jax
import jax.numpy as jnp
from jax import lax
from jax.experimental import pallas as pl
from jax.experimental.pallas import tpu as pltpu
from jax.experimental.pallas import tpu_sc as plsc

N = 10000
E = 320000
D = 128
DOUT = 16
NC = 2
NS = 16
L = 128
NB = 160
PER_SUB = NB * L
ACC_ROWS = 10240
ZCHUNK = ACC_ROWS // NS
IB = 32
RB = 1000
NRB = N // RB


@functools.cache
def _mesh():
    return plsc.VectorSubcoreMesh(core_axis_name="c", subcore_axis_name="s",
                                  num_cores=NC, num_subcores=NS)


def _sc_deg_body(dst_hbm, ones_hbm, zeros_hbm, out_hbm, dst_v, rows, acc):
    c = lax.axis_index("c")
    s = lax.axis_index("s")
    w = c * NS + s
    pltpu.sync_copy(zeros_hbm, rows)
    for k in range(ZCHUNK // L):
        pltpu.sync_copy(rows, acc.at[pl.ds(s * ZCHUNK + k * L, L)])
    pltpu.sync_copy(ones_hbm, rows)
    plsc.subcore_barrier()

    def outer(t, carry):
        pltpu.sync_copy(dst_hbm.at[pl.ds(w * NB + t * IB, IB)], dst_v)

        def body(j, carry2):
            pltpu.sync_copy(rows, acc.at[dst_v.at[j]], add=True)
            return carry2

        lax.fori_loop(0, IB, body, 0)
        return carry

    lax.fori_loop(0, NB // IB, outer, 0)
    plsc.subcore_barrier()
    for k in range(ZCHUNK // L):
        pltpu.sync_copy(acc.at[pl.ds(s * ZCHUNK + k * L, L)], rows)
        pltpu.sync_copy(rows, out_hbm.at[pl.ds(c * ACC_ROWS + s * ZCHUNK + k * L, L)])


@functools.cache
def _sc_deg_call():
    return pl.kernel(
        _sc_deg_body,
        out_type=jax.ShapeDtypeStruct((NC * ACC_ROWS, D), jnp.float32),
        mesh=_mesh(),
        scratch_types=[
            pltpu.VMEM((IB, L), jnp.int32),
            pltpu.VMEM((L, D), jnp.float32),
            pltpu.VMEM_SHARED((ACC_ROWS, D), jnp.float32),
        ],
    )


def _sc_deg(*args):
    return _sc_deg_call()(*args)


def _sc_prop_body(src_hbm, dst_hbm, g_hbm, zeros_hbm, out_hbm,
                  src_v, dst_v, rows, acc, sem):
    c = lax.axis_index("c")
    s = lax.axis_index("s")
    w = c * NS + s
    pltpu.sync_copy(zeros_hbm, rows)
    for k in range(ZCHUNK // L):
        pltpu.sync_copy(rows, acc.at[pl.ds(s * ZCHUNK + k * L, L)])
    plsc.subcore_barrier()

    def outer(t, carry):
        pltpu.sync_copy(src_hbm.at[pl.ds(w * NB + t * IB, IB)], src_v)
        pltpu.sync_copy(dst_hbm.at[pl.ds(w * NB + t * IB, IB)], dst_v)

        def body(j, carry2):
            pltpu.async_copy(g_hbm.at[src_v.at[j]], rows, sem).wait()
            pltpu.sync_copy(rows, acc.at[dst_v.at[j]], add=True)
            return carry2

        lax.fori_loop(0, IB, body, 0)
        return carry

    lax.fori_loop(0, NB // IB, outer, 0)
    plsc.subcore_barrier()
    for k in range(ZCHUNK // L):
        pltpu.sync_copy(acc.at[pl.ds(s * ZCHUNK + k * L, L)], rows)
        pltpu.sync_copy(rows, out_hbm.at[pl.ds(c * ACC_ROWS + s * ZCHUNK + k * L, L)])


@functools.cache
def _sc_prop_call():
    return pl.kernel(
        _sc_prop_body,
        out_type=jax.ShapeDtypeStruct((NC * ACC_ROWS, D), jnp.float32),
        mesh=_mesh(),
        scratch_types=[
            pltpu.VMEM((IB, L), jnp.int32),
            pltpu.VMEM((IB, L), jnp.int32),
            pltpu.VMEM((L, D), jnp.float32),
            pltpu.VMEM_SHARED((ACC_ROWS, D), jnp.float32),
            pltpu.SemaphoreType.DMA,
        ],
    )


def _sc_prop(*args):
    return _sc_prop_call()(*args)


def _dinv(deg_ref):
    return lax.rsqrt(deg_ref[0, :, 0:1] + 1.0)


def _elu(v):
    return jnp.where(v > 0.0, v, jnp.exp(v) - 1.0)


def _tc_in_body(x_ref, w_ref, deg_ref, g_ref):
    dinv = _dinv(deg_ref)
    g_ref[0] = jnp.dot(x_ref[0], w_ref[0],
                       preferred_element_type=jnp.float32) * dinv


def _tc_mid_body(s_ref, g_ref, deg_ref, b_ref, w_ref, out_ref):
    dinv = _dinv(deg_ref)
    h = _elu(dinv * (s_ref[0] + g_ref[0]) + b_ref[0])
    out_ref[0] = jnp.dot(h, w_ref[0], preferred_element_type=jnp.float32) * dinv


def _tc_out_body(s_ref, g_ref, deg_ref, b_ref, fcw_ref, fcb_ref, out_ref):
    dinv = _dinv(deg_ref)
    h = _elu(dinv * (s_ref[0] + g_ref[0]) + b_ref[0])
    logits = jnp.dot(h, fcw_ref[0], preferred_element_type=jnp.float32) + fcb_ref[0]
    m = jnp.max(logits, axis=1, keepdims=True)
    e = jnp.exp(logits - m)
    out_ref[0] = e / jnp.sum(e, axis=1, keepdims=True)


def _row_spec(width):
    return pl.BlockSpec((1, RB, width), lambda b, r: (b, r, 0))


def _mat_spec(rows, cols):
    return pl.BlockSpec((1, rows, cols), lambda b, r: (b, 0, 0))


def _vec_spec(width):
    return pl.BlockSpec((1, 1, width), lambda b, r: (b, 0, 0))


_tc_in = pl.pallas_call(
    _tc_in_body,
    grid=(NC, NRB),
    in_specs=[_row_spec(D), _mat_spec(D, D), _row_spec(D)],
    out_specs=_row_spec(D),
    out_shape=jax.ShapeDtypeStruct((NC, N, D), jnp.float32),
)

_tc_mid = pl.pallas_call(
    _tc_mid_body,
    grid=(NC, NRB),
    in_specs=[_row_spec(D), _row_spec(D), _row_spec(D), _vec_spec(D),
              _mat_spec(D, D)],
    out_specs=_row_spec(D),
    out_shape=jax.ShapeDtypeStruct((NC, N, D), jnp.float32),
)

_tc_out = pl.pallas_call(
    _tc_out_body,
    grid=(NC, NRB),
    in_specs=[_row_spec(D), _row_spec(D), _row_spec(D), _vec_spec(D),
              _mat_spec(D, DOUT), _vec_spec(DOUT)],
    out_specs=_row_spec(DOUT),
    out_shape=jax.ShapeDtypeStruct((NC, N, DOUT), jnp.float32),
)


def kernel(x, edge_index_x, y, edge_index_y, Wx1, bx1, Wx2, bx2, fcWx, fcbx,
           Wy1, by1, Wy2, by2, fcWy, fcby):
    pad = NS * PER_SUB - E

    def stage(ei, row_off):
        src = ei[0].astype(jnp.int32)
        dst = ei[1].astype(jnp.int32)
        srcp = jnp.concatenate([src + row_off,
                                jnp.full((pad,), row_off, jnp.int32)])
        dstp = jnp.concatenate([dst, jnp.full((pad,), N, jnp.int32)])
        return srcp, dstp

    sx, dx = stage(edge_index_x, 0)
    sy, dy = stage(edge_index_y, N)
    src_all = jnp.stack([sx, sy]).reshape(NC * NS * NB, L)
    dst_all = jnp.stack([dx, dy]).reshape(NC * NS * NB, L)
    zrows = jnp.zeros((L, D), jnp.float32)
    orows = jnp.ones((L, D), jnp.float32)

    deg = _sc_deg(dst_all, orows, zrows).reshape(NC, ACC_ROWS, D)

    x_all = jnp.stack([x, y])
    W1 = jnp.stack([Wx1, Wy1])
    b1 = jnp.stack([bx1, by1]).reshape(NC, 1, D)
    W2 = jnp.stack([Wx2, Wy2])
    b2 = jnp.stack([bx2, by2]).reshape(NC, 1, D)
    fcW = jnp.stack([fcWx, fcWy])
    fcb = jnp.stack([fcbx, fcby]).reshape(NC, 1, DOUT)

    g1 = _tc_in(x_all, W1, deg)
    s1 = _sc_prop(src_all, dst_all, g1.reshape(NC * N, D),
                  zrows).reshape(NC, ACC_ROWS, D)
    g2 = _tc_mid(s1, g1, deg, b1, W2)
    s2 = _sc_prop(src_all, dst_all, g2.reshape(NC * N, D),
                  zrows).reshape(NC, ACC_ROWS, D)
    out = _tc_out(s2, g2, deg, b2, fcW, fcb)
    return out[0], out[1]


# speedup vs baseline: 10.9872x; 1.1802x over previous
import functools

import jax
import jax.numpy as jnp
from jax import lax
from jax.experimental import pallas as pl
from jax.experimental.pallas import tpu as pltpu
from jax.experimental.pallas import tpu_sc as plsc

N = 10000
E = 320000
D = 128
DOUT = 16
NC = 2
NS = 16
L = 128
NB = 160
PER_SUB = NB * L
ACC_ROWS = 10240
ZCHUNK = ACC_ROWS // NS
IB = 32
RB = 1000
NRB = N // RB


@functools.cache
def _mesh():
    return plsc.VectorSubcoreMesh(core_axis_name="c", subcore_axis_name="s",
                                  num_cores=NC, num_subcores=NS)


def _sc_deg_body(dst_hbm, ones_hbm, zeros_hbm, out_hbm, dst_v, rows, acc):
    c = lax.axis_index("c")
    s = lax.axis_index("s")
    w = c * NS + s
    pltpu.sync_copy(zeros_hbm, rows)
    for k in range(ZCHUNK // L):
        pltpu.sync_copy(rows, acc.at[pl.ds(s * ZCHUNK + k * L, L)])
    pltpu.sync_copy(ones_hbm, rows)
    plsc.subcore_barrier()

    def outer(t, carry):
        pltpu.sync_copy(dst_hbm.at[pl.ds(w * NB + t * IB, IB)], dst_v)

        def body(j, carry2):
            pltpu.sync_copy(rows, acc.at[dst_v.at[j]], add=True)
            return carry2

        lax.fori_loop(0, IB, body, 0)
        return carry

    lax.fori_loop(0, NB // IB, outer, 0)
    plsc.subcore_barrier()
    for k in range(ZCHUNK // L):
        pltpu.sync_copy(acc.at[pl.ds(s * ZCHUNK + k * L, L)], rows)
        pltpu.sync_copy(rows, out_hbm.at[pl.ds(c * ACC_ROWS + s * ZCHUNK + k * L, L)])


@functools.cache
def _sc_deg_call():
    return pl.kernel(
        _sc_deg_body,
        out_type=jax.ShapeDtypeStruct((NC * ACC_ROWS, D), jnp.float32),
        mesh=_mesh(),
        scratch_types=[
            pltpu.VMEM((IB, L), jnp.int32),
            pltpu.VMEM((L, D), jnp.float32),
            pltpu.VMEM_SHARED((ACC_ROWS, D), jnp.float32),
        ],
    )


def _sc_deg(*args):
    return _sc_deg_call()(*args)


def _sc_prop_body(src_hbm, dst_hbm, g_hbm, zeros_hbm, out_hbm,
                  src_v, dst_v, rows_a, rows_b, acc, sem_a, sem_b):
    c = lax.axis_index("c")
    s = lax.axis_index("s")
    w = c * NS + s
    pltpu.sync_copy(zeros_hbm, rows_a)
    for k in range(ZCHUNK // L):
        pltpu.sync_copy(rows_a, acc.at[pl.ds(s * ZCHUNK + k * L, L)])
    plsc.subcore_barrier()

    def gather(j, buf, sem):
        pltpu.async_copy(g_hbm.at[src_v.at[j]], buf, sem)

    def wait(buf, sem):
        pltpu.make_async_copy(g_hbm.at[src_v.at[0]], buf, sem).wait()

    def scatter(j, buf):
        pltpu.sync_copy(buf, acc.at[dst_v.at[j]], add=True)

    def outer(t, carry):
        pltpu.sync_copy(src_hbm.at[pl.ds(w * NB + t * IB, IB)], src_v)
        pltpu.sync_copy(dst_hbm.at[pl.ds(w * NB + t * IB, IB)], dst_v)
        gather(0, rows_a, sem_a)

        def body(i, carry2):
            gather(2 * i + 1, rows_b, sem_b)
            wait(rows_a, sem_a)
            scatter(2 * i, rows_a)
            gather(2 * i + 2, rows_a, sem_a)
            wait(rows_b, sem_b)
            scatter(2 * i + 1, rows_b)
            return carry2

        lax.fori_loop(0, IB // 2 - 1, body, 0)
        gather(IB - 1, rows_b, sem_b)
        wait(rows_a, sem_a)
        scatter(IB - 2, rows_a)
        wait(rows_b, sem_b)
        scatter(IB - 1, rows_b)
        return carry

    lax.fori_loop(0, NB // IB, outer, 0)
    plsc.subcore_barrier()
    for k in range(ZCHUNK // L):
        pltpu.sync_copy(acc.at[pl.ds(s * ZCHUNK + k * L, L)], rows_a)
        pltpu.sync_copy(rows_a, out_hbm.at[pl.ds(c * ACC_ROWS + s * ZCHUNK + k * L, L)])


@functools.cache
def _sc_prop_call():
    return pl.kernel(
        _sc_prop_body,
        out_type=jax.ShapeDtypeStruct((NC * ACC_ROWS, D), jnp.float32),
        mesh=_mesh(),
        scratch_types=[
            pltpu.VMEM((IB, L), jnp.int32),
            pltpu.VMEM((IB, L), jnp.int32),
            pltpu.VMEM((L, D), jnp.float32),
            pltpu.VMEM((L, D), jnp.float32),
            pltpu.VMEM_SHARED((ACC_ROWS, D), jnp.float32),
            pltpu.SemaphoreType.DMA,
            pltpu.SemaphoreType.DMA,
        ],
    )


def _sc_prop(*args):
    return _sc_prop_call()(*args)


def _dinv(deg_ref):
    return lax.rsqrt(deg_ref[0, :, 0:1] + 1.0)


def _elu(v):
    return jnp.where(v > 0.0, v, jnp.exp(v) - 1.0)


def _tc_in_body(x_ref, w_ref, deg_ref, g_ref):
    dinv = _dinv(deg_ref)
    g_ref[0] = jnp.dot(x_ref[0], w_ref[0],
                       preferred_element_type=jnp.float32) * dinv


def _tc_mid_body(s_ref, g_ref, deg_ref, b_ref, w_ref, out_ref):
    dinv = _dinv(deg_ref)
    h = _elu(dinv * (s_ref[0] + g_ref[0]) + b_ref[0])
    out_ref[0] = jnp.dot(h, w_ref[0], preferred_element_type=jnp.float32) * dinv


def _tc_out_body(s_ref, g_ref, deg_ref, b_ref, fcw_ref, fcb_ref, out_ref):
    dinv = _dinv(deg_ref)
    h = _elu(dinv * (s_ref[0] + g_ref[0]) + b_ref[0])
    logits = jnp.dot(h, fcw_ref[0], preferred_element_type=jnp.float32) + fcb_ref[0]
    m = jnp.max(logits, axis=1, keepdims=True)
    e = jnp.exp(logits - m)
    out_ref[0] = e / jnp.sum(e, axis=1, keepdims=True)


def _row_spec(width):
    return pl.BlockSpec((1, RB, width), lambda b, r: (b, r, 0))


def _mat_spec(rows, cols):
    return pl.BlockSpec((1, rows, cols), lambda b, r: (b, 0, 0))


def _vec_spec(width):
    return pl.BlockSpec((1, 1, width), lambda b, r: (b, 0, 0))


_tc_in = pl.pallas_call(
    _tc_in_body,
    grid=(NC, NRB),
    in_specs=[_row_spec(D), _mat_spec(D, D), _row_spec(D)],
    out_specs=_row_spec(D),
    out_shape=jax.ShapeDtypeStruct((NC, N, D), jnp.float32),
)

_tc_mid = pl.pallas_call(
    _tc_mid_body,
    grid=(NC, NRB),
    in_specs=[_row_spec(D), _row_spec(D), _row_spec(D), _vec_spec(D),
              _mat_spec(D, D)],
    out_specs=_row_spec(D),
    out_shape=jax.ShapeDtypeStruct((NC, N, D), jnp.float32),
)

_tc_out = pl.pallas_call(
    _tc_out_body,
    grid=(NC, NRB),
    in_specs=[_row_spec(D), _row_spec(D), _row_spec(D), _vec_spec(D),
              _mat_spec(D, DOUT), _vec_spec(DOUT)],
    out_specs=_row_spec(DOUT),
    out_shape=jax.ShapeDtypeStruct((NC, N, DOUT), jnp.float32),
)


def kernel(x, edge_index_x, y, edge_index_y, Wx1, bx1, Wx2, bx2, fcWx, fcbx,
           Wy1, by1, Wy2, by2, fcWy, fcby):
    pad = NS * PER_SUB - E

    def stage(ei, row_off):
        src = ei[0].astype(jnp.int32)
        dst = ei[1].astype(jnp.int32)
        srcp = jnp.concatenate([src + row_off,
                                jnp.full((pad,), row_off, jnp.int32)])
        dstp = jnp.concatenate([dst, jnp.full((pad,), N, jnp.int32)])
        return srcp, dstp

    sx, dx = stage(edge_index_x, 0)
    sy, dy = stage(edge_index_y, N)
    src_all = jnp.stack([sx, sy]).reshape(NC * NS * NB, L)
    dst_all = jnp.stack([dx, dy]).reshape(NC * NS * NB, L)
    zrows = jnp.zeros((L, D), jnp.float32)
    orows = jnp.ones((L, D), jnp.float32)

    deg = _sc_deg(dst_all, orows, zrows).reshape(NC, ACC_ROWS, D)

    x_all = jnp.stack([x, y])
    W1 = jnp.stack([Wx1, Wy1])
    b1 = jnp.stack([bx1, by1]).reshape(NC, 1, D)
    W2 = jnp.stack([Wx2, Wy2])
    b2 = jnp.stack([bx2, by2]).reshape(NC, 1, D)
    fcW = jnp.stack([fcWx, fcWy])
    fcb = jnp.stack([fcbx, fcby]).reshape(NC, 1, DOUT)

    g1 = _tc_in(x_all, W1, deg)
    s1 = _sc_prop(src_all, dst_all, g1.reshape(NC * N, D),
                  zrows).reshape(NC, ACC_ROWS, D)
    g2 = _tc_mid(s1, g1, deg, b1, W2)
    s2 = _sc_prop(src_all, dst_all, g2.reshape(NC * N, D),
                  zrows).reshape(NC, ACC_ROWS, D)
    out = _tc_out(s2, g2, deg, b2, fcW, fcb)
    return out[0], out[1]
